# Initial kernel scaffold; baseline (speedup 1.0000x reference)
import jax
import jax.numpy as jnp
from jax.experimental import pallas as pl


def kernel(T, mass, L, kap_conductivity, edge_index, edge_A, edge_L, edge_conductivity, static_heat, specific_heat_capacity, time_step):
    raise NotImplementedError("write your pallas kernel here")


# speedup vs baseline: 117.8818x; 117.8818x over previous
import functools

import jax
import jax.numpy as jnp
from jax import lax
from jax.experimental import pallas as pl
from jax.experimental.pallas import tpu as pltpu
from jax.experimental.pallas import tpu_sc as plsc

_N = 50000
_E = 800000
_S = 30
_T_LIQUID4 = 1.9 ** 4

_NT = 16
_NP = 51200
_NR = 3200
_RR = _NR // 128
_ROWS = _NP // 128
_EPT = _E // _NT
_C = 2000
_NCH = _EPT // _C


def _heat_body(T_hbm, src_hbm, dst_hbm, coef_hbm, kl_hbm, g_hbm, sh_hbm,
               Tst, Pst, Fst,
               T_loc, acc, src_b, dst_b, coef_b, flux_b,
               kl_b, g_b, heat_b, pw_b, sh_b, idx_v,
               acc_sh, T_sh):
    sid = lax.axis_index("s")
    base = pl.multiple_of(sid * _NR, 8)
    rbase = sid * _RR
    ebase = pl.multiple_of(sid * _EPT, 8)
    zv = jnp.zeros((16,), jnp.float32)
    lane = lax.iota(jnp.int32, 16)

    pltpu.sync_copy(T_hbm, T_loc)
    pltpu.sync_copy(kl_hbm.at[pl.ds(rbase, _RR)], kl_b)
    pltpu.sync_copy(g_hbm.at[pl.ds(rbase, _RR)], g_b)
    pltpu.sync_copy(sh_hbm, sh_b)
    sh_v = sh_b[...]

    def _fill_idx(v, carry):
        idx_v[v, :] = v * 16 + lane
        return carry
    lax.fori_loop(0, _ROWS // 16, _fill_idx, 0)

    def _zheat(r, carry):
        for i in range(8):
            heat_b[r, pl.ds(i * 16, 16)] = zv
        return carry
    lax.fori_loop(0, _RR, _zheat, 0)
    pltpu.sync_copy(heat_b, acc_sh.at[pl.ds(rbase, _RR)])
    pltpu.sync_copy(heat_b, Pst.at[pl.ds(rbase, _RR)])
    pltpu.sync_copy(T_loc.at[pl.ds(base, _NR)], Tst.at[pl.ds(base, _NR)])
    plsc.subcore_barrier()

    def _step(s, carry):
        def _zacc(r, c2):
            for i in range(8):
                acc[r, pl.ds(i * 16, 16)] = zv
            return c2
        lax.fori_loop(0, _ROWS, _zacc, 0)

        def _chunk(c, c2):
            eo = pl.multiple_of(ebase + c * _C, 8)
            pltpu.sync_copy(src_hbm.at[pl.ds(eo, _C)], src_b)
            pltpu.sync_copy(dst_hbm.at[pl.ds(eo, _C)], dst_b)
            pltpu.sync_copy(coef_hbm.at[pl.ds(eo, _C)], coef_b)

            def _edge(v, c3):
                o = v * 16
                si = src_b[pl.ds(o, 16)]
                di = dst_b[pl.ds(o, 16)]
                ts = plsc.load_gather(T_loc, [si])
                td = plsc.load_gather(T_loc, [di])
                fx = coef_b[pl.ds(o, 16)] * (ts - td)
                flux_b[pl.ds(o, 16)] = fx
                dr = lax.shift_right_logical(di, 7)
                dc = lax.bitwise_and(di, 127)
                sr = lax.shift_right_logical(si, 7)
                sc = lax.bitwise_and(si, 127)
                plsc.addupdate_scatter(acc, [dr, dc], fx)
                plsc.addupdate_scatter(acc, [sr, sc], -fx)
                return c3
            lax.fori_loop(0, _C // 16, _edge, 0)

            fo = pl.multiple_of(s * _E + eo, 8)
            pltpu.sync_copy(flux_b, Fst.at[pl.ds(fo, _C)])
            return c2
        lax.fori_loop(0, _NCH, _chunk, 0)

        for v in range(_ROWS // 16):
            pltpu.sync_copy(acc.at[pl.ds(v * 16, 16)],
                            acc_sh.at[idx_v.at[v]], add=True)
        plsc.subcore_barrier()

        pltpu.sync_copy(acc_sh.at[pl.ds(rbase, _RR)], heat_b)

        def _upd(r, c2):
            for i in range(8):
                o = base + r * 128 + i * 16
                li = pl.ds(i * 16, 16)
                tv = T_loc[pl.ds(o, 16)]
                t2 = tv * tv
                t4 = t2 * t2
                pw = kl_b[r, li] * (t4 - _T_LIQUID4)
                tn = tv + (heat_b[r, li] + sh_v - pw) * g_b[r, li]
                T_loc[pl.ds(o, 16)] = tn
                pw_b[r, li] = pw
            return c2
        lax.fori_loop(0, _RR, _upd, 0)

        lax.fori_loop(0, _RR, _zheat, 0)
        pltpu.sync_copy(heat_b, acc_sh.at[pl.ds(rbase, _RR)])

        pltpu.sync_copy(T_loc.at[pl.ds(base, _NR)], T_sh.at[pl.ds(base, _NR)])
        ot = pl.multiple_of((s + 1) * _NP + base, 8)
        pltpu.sync_copy(T_loc.at[pl.ds(base, _NR)], Tst.at[pl.ds(ot, _NR)])
        pltpu.sync_copy(pw_b, Pst.at[pl.ds((s + 1) * _ROWS + rbase, _RR)])
        plsc.subcore_barrier()

        pltpu.sync_copy(T_sh, T_loc)
        plsc.subcore_barrier()
        return carry

    lax.fori_loop(0, _S, _step, 0)


@jax.jit
def _run(T_pad, src, dst, coef, kl2, g2, sh16):
    mesh = plsc.VectorSubcoreMesh(
        core_axis_name="c", subcore_axis_name="s", num_cores=1)
    f = functools.partial(
        pl.kernel,
        out_type=(
            jax.ShapeDtypeStruct(((_S + 1) * _NP,), jnp.float32),
            jax.ShapeDtypeStruct(((_S + 1) * _ROWS, 128), jnp.float32),
            jax.ShapeDtypeStruct((_S * _E,), jnp.float32),
        ),
        mesh=mesh,
        compiler_params=pltpu.CompilerParams(
            needs_layout_passes=False, use_tc_tiling_on_sc=False),
        scratch_types=[
            pltpu.VMEM((_NP,), jnp.float32),
            pltpu.VMEM((_ROWS, 128), jnp.float32),
            pltpu.VMEM((_C,), jnp.int32),
            pltpu.VMEM((_C,), jnp.int32),
            pltpu.VMEM((_C,), jnp.float32),
            pltpu.VMEM((_C,), jnp.float32),
            pltpu.VMEM((_RR, 128), jnp.float32),
            pltpu.VMEM((_RR, 128), jnp.float32),
            pltpu.VMEM((_RR, 128), jnp.float32),
            pltpu.VMEM((_RR, 128), jnp.float32),
            pltpu.VMEM((16,), jnp.float32),
            pltpu.VMEM((_ROWS // 16, 16), jnp.int32),
            pltpu.VMEM_SHARED((_ROWS, 128), jnp.float32),
            pltpu.VMEM_SHARED((_NP,), jnp.float32),
        ],
    )(_heat_body)
    return f(T_pad, src, dst, coef, kl2, g2, sh16)


def kernel(T, mass, L, kap_conductivity, edge_index, edge_A, edge_L,
           edge_conductivity, static_heat, specific_heat_capacity, time_step):
    src = edge_index[0]
    dst = edge_index[1]
    coef = edge_conductivity * edge_A / edge_L
    cap = mass * specific_heat_capacity[0] + 1e-6
    dt = time_step[0] * 1e-3
    pad = _NP - _N
    T_pad = jnp.pad(T, (0, pad), constant_values=1.9)
    kl2 = jnp.pad(kap_conductivity * L, (0, pad)).reshape(_ROWS, 128)
    g2 = jnp.pad(dt / cap, (0, pad)).reshape(_ROWS, 128)
    sh16 = jnp.full((16,), static_heat[0] / _N, dtype=jnp.float32)

    Tst_p, Pst_p, Fst = _run(T_pad, src, dst, coef, kl2, g2, sh16)

    Tst = Tst_p.reshape(_S + 1, _NP)[:, :_N].reshape(-1)
    Pst = Pst_p.reshape(_S + 1, _NP)[:, :_N].reshape(-1)
    times = jnp.arange(_S + 1, dtype=jnp.float32) * time_step[0]
    return (times, Tst, Pst, Fst)
